# baseline (speedup 1.0000x reference)
import functools

import jax
import jax.numpy as jnp
from jax import lax
from jax.experimental import pallas as pl
from jax.experimental.pallas import tpu as pltpu
from jax.experimental.pallas import tpu_sc as plsc

_BATCH = 4096
_HIST = 200
_D = 64
_DP = 128
_V = 1000000
_B = _BATCH * _HIST
_NC = 2
_NS = 16
_NW = _NC * _NS
_BB = _BATCH // _NW
_NBUF = 4
_L = 16


def _make_kernel():
    mesh = plsc.VectorSubcoreMesh(core_axis_name="c", subcore_axis_name="s")

    @functools.partial(
        pl.kernel,
        out_type=jax.ShapeDtypeStruct(
            (_HIST, _D // 8, _BATCH // _BB, 8 * _DP), jnp.float32),
        mesh=mesh,
        scratch_types=(
            [pltpu.VMEM((_BB * _HIST,), jnp.int32),
             pltpu.VMEM((_HIST * _BB,), jnp.int32),
             pltpu.VMEM((_NBUF, _BB, _D), jnp.float32),
             pltpu.VMEM((_NBUF, _D * _BB), jnp.float32)]
            + [pltpu.SemaphoreType.DMA] * (2 * _NBUF)
        ),
        compiler_params=pltpu.CompilerParams(
            use_tc_tiling_on_sc=False, needs_layout_passes=False),
    )
    def emb(idx_hbm, table_hbm, out_hbm, idxb_v, idxt_v, rows_v, trows_v,
            *sems):
        gsems = sems[:_NBUF]
        osems = sems[_NBUF:]
        wid = lax.axis_index("s") * _NC + lax.axis_index("c")
        b0 = wid * _BB

        pltpu.sync_copy(idx_hbm.at[pl.ds(b0 * _HIST, _BB * _HIST)], idxb_v)

        lanes = lax.iota(jnp.int32, _L)
        rowsel = [_HIST * (j * _L + lanes) for j in range(_BB // _L)]
        lanesel = [j * _L + lanes for j in range(_BB // _L)]

        @pl.loop(0, _HIST)
        def _tidx(h):
            for j in range(_BB // _L):
                v = plsc.load_gather(idxb_v, [rowsel[j] + h])
                idxt_v[pl.ds(h * _BB + j * _L, _L)] = v

        def start_gather(h, b):
            pltpu.async_copy(
                table_hbm.at[idxt_v.at[pl.ds(h * _BB, _BB)]],
                rows_v.at[b], gsems[b])

        def wait_gather(b):
            pltpu.make_async_copy(
                table_hbm.at[idxt_v.at[pl.ds(0, _BB)]],
                rows_v.at[b], gsems[b]).wait()

        def transpose(b):
            @pl.loop(0, _D)
            def _t(d):
                dcol = jnp.broadcast_to(d, (_L,)).astype(jnp.int32)
                for j in range(_BB // _L):
                    v = plsc.load_gather(rows_v.at[b], [lanesel[j], dcol])
                    trows_v[b, pl.ds(d * _BB + j * _L, _L)] = v

        def start_store(h, b):
            for rb in range(_D // 8):
                pltpu.async_copy(
                    trows_v.at[b, pl.ds(rb * 8 * _DP, 8 * _DP)],
                    out_hbm.at[h, rb, wid], osems[b])

        def wait_store(b):
            for rb in range(_D // 8):
                pltpu.make_async_copy(
                    trows_v.at[b, pl.ds(rb * 8 * _DP, 8 * _DP)],
                    out_hbm.at[0, rb, wid], osems[b]).wait()

        for b in range(_NBUF):
            start_gather(b, b)
        for b in range(_NBUF):
            wait_gather(b)
            transpose(b)
            start_store(b, b)
            start_gather(b + _NBUF, b)

        @pl.loop(1, _HIST // _NBUF - 1)
        def _grp(g):
            i0 = g * _NBUF
            for b in range(_NBUF):
                wait_gather(b)
                wait_store(b)
                transpose(b)
                start_store(i0 + b, b)
                start_gather(i0 + b + _NBUF, b)

        i0 = _HIST - _NBUF
        for b in range(_NBUF):
            wait_gather(b)
            wait_store(b)
            transpose(b)
            start_store(i0 + b, b)
        for b in range(_NBUF):
            wait_store(b)

    return emb


_emb = _make_kernel()


@jax.jit
def kernel(x, table):
    idx2 = (x.astype(jnp.int32) * 2).reshape(_B)
    tab2m = jnp.pad(table, ((0, 0), (0, _DP - _D))).reshape(2 * _V, _D)
    out4 = _emb(idx2, tab2m)
    out5 = out4.reshape(_HIST, _D // 8, _BATCH // _BB, 8, _DP)
    return out5.transpose(2, 4, 0, 1, 3).reshape(_BATCH, _HIST, _D)


# speedup vs baseline: 1.4369x; 1.4369x over previous
import functools

import jax
import jax.numpy as jnp
from jax import lax
from jax.experimental import pallas as pl
from jax.experimental.pallas import tpu as pltpu
from jax.experimental.pallas import tpu_sc as plsc

_BATCH = 4096
_HIST = 200
_D = 64
_DP = 128
_V = 1000000
_B = _BATCH * _HIST
_NC = 2
_NS = 16
_NW = _NC * _NS
_BB = _BATCH // _NW
_NBUF = 4
_L = 16


def _make_kernel():
    mesh = plsc.VectorSubcoreMesh(core_axis_name="c", subcore_axis_name="s")

    @functools.partial(
        pl.kernel,
        out_type=jax.ShapeDtypeStruct(
            (_HIST, _D // 8, _BATCH // _BB, 8 * _DP), jnp.float32),
        mesh=mesh,
        scratch_types=(
            [pltpu.VMEM((_BB * _HIST,), jnp.int32),
             pltpu.VMEM((_HIST * _BB,), jnp.int32),
             pltpu.VMEM((_NBUF, _BB, _D), jnp.float32),
             pltpu.VMEM((_NBUF, _D * _BB), jnp.float32)]
            + [pltpu.SemaphoreType.DMA] * (2 * _NBUF)
        ),
        compiler_params=pltpu.CompilerParams(
            use_tc_tiling_on_sc=False, needs_layout_passes=False),
    )
    def emb(idx_hbm, table_hbm, out_hbm, idxb_v, idxt_v, rows_v, trows_v,
            *sems):
        gsems = sems[:_NBUF]
        osems = sems[_NBUF:]
        wid = lax.axis_index("s") * _NC + lax.axis_index("c")
        b0 = wid * _BB

        pltpu.sync_copy(idx_hbm.at[pl.ds(b0 * _HIST, _BB * _HIST)], idxb_v)

        lanes = lax.iota(jnp.int32, _L)
        rowsel = [_HIST * (j * _L + lanes) for j in range(_BB // _L)]
        lanesel = [j * _L + lanes for j in range(_BB // _L)]

        @plsc.parallel_loop(0, _HIST, unroll=4)
        def _tidx(h):
            for j in range(_BB // _L):
                v = plsc.load_gather(idxb_v, [rowsel[j] + h])
                idxt_v[pl.ds(h * _BB + j * _L, _L)] = v

        def start_gather(h, b):
            pltpu.async_copy(
                table_hbm.at[idxt_v.at[pl.ds(h * _BB, _BB)]],
                rows_v.at[b], gsems[b])

        def wait_gather(b):
            pltpu.make_async_copy(
                table_hbm.at[idxt_v.at[pl.ds(0, _BB)]],
                rows_v.at[b], gsems[b]).wait()

        def transpose(b):
            @plsc.parallel_loop(0, _D, unroll=8)
            def _t(d):
                dcol = jnp.broadcast_to(d, (_L,)).astype(jnp.int32)
                for j in range(_BB // _L):
                    v = plsc.load_gather(rows_v.at[b], [lanesel[j], dcol])
                    trows_v[b, pl.ds(d * _BB + j * _L, _L)] = v

        def start_store(h, b):
            for rb in range(_D // 8):
                pltpu.async_copy(
                    trows_v.at[b, pl.ds(rb * 8 * _DP, 8 * _DP)],
                    out_hbm.at[h, rb, wid], osems[b])

        def wait_store(b):
            for rb in range(_D // 8):
                pltpu.make_async_copy(
                    trows_v.at[b, pl.ds(rb * 8 * _DP, 8 * _DP)],
                    out_hbm.at[0, rb, wid], osems[b]).wait()

        for b in range(_NBUF):
            start_gather(b, b)
        for b in range(_NBUF):
            wait_gather(b)
            transpose(b)
            start_store(b, b)
            start_gather(b + _NBUF, b)

        @pl.loop(1, _HIST // _NBUF - 1)
        def _grp(g):
            i0 = g * _NBUF
            for b in range(_NBUF):
                wait_gather(b)
                wait_store(b)
                transpose(b)
                start_store(i0 + b, b)
                start_gather(i0 + b + _NBUF, b)

        i0 = _HIST - _NBUF
        for b in range(_NBUF):
            wait_gather(b)
            wait_store(b)
            transpose(b)
            start_store(i0 + b, b)
        for b in range(_NBUF):
            wait_store(b)

    return emb


_emb = _make_kernel()


@jax.jit
def kernel(x, table):
    idx2 = (x.astype(jnp.int32) * 2).reshape(_B)
    tab2m = jnp.pad(table, ((0, 0), (0, _DP - _D))).reshape(2 * _V, _D)
    out4 = _emb(idx2, tab2m)
    out5 = out4.reshape(_HIST, _D // 8, _BATCH // _BB, 8, _DP)
    return out5.transpose(2, 4, 0, 1, 3).reshape(_BATCH, _HIST, _D)


# speedup vs baseline: 2.0455x; 1.4235x over previous
import functools

import jax
import jax.numpy as jnp
from jax import lax
from jax.experimental import pallas as pl
from jax.experimental.pallas import tpu as pltpu
from jax.experimental.pallas import tpu_sc as plsc

_BATCH = 4096
_HIST = 200
_D = 64
_DP = 128
_V = 1000000
_B = _BATCH * _HIST
_NC = 2
_NS = 16
_NW = _NC * _NS
_BPW = _B // _NW
_CH = _HIST
_NBUF = 8
_NCHUNK = _BPW // _CH
_NGRP = _NCHUNK // _NBUF - 1


def _make_kernel():
    mesh = plsc.VectorSubcoreMesh(core_axis_name="c", subcore_axis_name="s")

    @functools.partial(
        pl.kernel,
        out_type=jax.ShapeDtypeStruct((_BATCH, _HIST, _DP), jnp.float32),
        mesh=mesh,
        scratch_types=(
            [pltpu.VMEM((_BPW,), jnp.int32),
             pltpu.VMEM((_NBUF, _CH, _D), jnp.float32)]
            + [pltpu.SemaphoreType.DMA] * (2 * _NBUF)
        ),
        compiler_params=pltpu.CompilerParams(use_tc_tiling_on_sc=False),
    )
    def emb(idx_hbm, table_hbm, out_hbm, idx_v, rows_v, *sems):
        gsems = sems[:_NBUF]
        osems = sems[_NBUF:]
        wid = lax.axis_index("s") * _NC + lax.axis_index("c")
        base = wid * _BPW
        b0 = wid * (_BATCH // _NW)

        pltpu.sync_copy(idx_hbm.at[pl.ds(base, _BPW)], idx_v)

        def start_gather(i, b):
            pltpu.async_copy(
                table_hbm.at[idx_v.at[pl.ds(i * _CH, _CH)]],
                rows_v.at[b], gsems[b])

        def wait_gather(b):
            pltpu.make_async_copy(
                table_hbm.at[idx_v.at[pl.ds(0, _CH)]],
                rows_v.at[b], gsems[b]).wait()

        def start_store(i, b):
            pltpu.async_copy(
                rows_v.at[b], out_hbm.at[b0 + i, :, pl.ds(0, _D)], osems[b])

        def wait_store(b):
            pltpu.make_async_copy(
                rows_v.at[b], out_hbm.at[b0, :, pl.ds(0, _D)], osems[b]).wait()

        for b in range(_NBUF):
            start_gather(b, b)

        @pl.loop(0, _NGRP)
        def _grp(g):
            i0 = g * _NBUF
            for b in range(_NBUF):
                wait_gather(b)
                start_store(i0 + b, b)
            for b in range(_NBUF):
                wait_store(b)
                start_gather(i0 + b + _NBUF, b)

        i0 = _NGRP * _NBUF
        for b in range(_NBUF):
            wait_gather(b)
            start_store(i0 + b, b)
        for b in range(_NBUF):
            wait_store(b)

    return emb


_emb = _make_kernel()


@jax.jit
def kernel(x, table):
    idx = x.reshape(_B).astype(jnp.int32) * 2
    tab2m = jnp.pad(table, ((0, 0), (0, _DP - _D))).reshape(2 * _V, _D)
    out128 = _emb(idx, tab2m)
    return out128[:, :, :_D]


# speedup vs baseline: 2.0465x; 1.0005x over previous
import functools

import jax
import jax.numpy as jnp
from jax import lax
from jax.experimental import pallas as pl
from jax.experimental.pallas import tpu as pltpu
from jax.experimental.pallas import tpu_sc as plsc

_BATCH = 4096
_HIST = 200
_D = 64
_DP = 128
_V = 1000000
_B = _BATCH * _HIST
_NC = 2
_NS = 16
_NW = _NC * _NS
_BPW = _B // _NW
_CH = 2 * _HIST
_NBUF = 4
_NCHUNK = _BPW // _CH
_NGRP = _NCHUNK // _NBUF - 1


def _make_kernel():
    mesh = plsc.VectorSubcoreMesh(core_axis_name="c", subcore_axis_name="s")

    @functools.partial(
        pl.kernel,
        out_type=jax.ShapeDtypeStruct((_BATCH, _HIST, _DP), jnp.float32),
        mesh=mesh,
        scratch_types=(
            [pltpu.VMEM((_BPW,), jnp.int32),
             pltpu.VMEM((_NBUF, 2, _HIST, _D), jnp.float32)]
            + [pltpu.SemaphoreType.DMA] * (2 * _NBUF)
        ),
        compiler_params=pltpu.CompilerParams(use_tc_tiling_on_sc=False),
    )
    def emb(idx_hbm, table_hbm, out_hbm, idx_v, rows_v, *sems):
        gsems = sems[:_NBUF]
        osems = sems[_NBUF:]
        wid = lax.axis_index("s") * _NC + lax.axis_index("c")
        base = wid * _BPW
        b0 = wid * (_BATCH // _NW)

        pltpu.sync_copy(idx_hbm.at[pl.ds(base, _BPW)], idx_v)

        def start_gather(i, b):
            for u in range(2):
                pltpu.async_copy(
                    table_hbm.at[idx_v.at[pl.ds((2 * i + u) * _HIST, _HIST)]],
                    rows_v.at[b, u], gsems[b])

        def wait_gather(b):
            for u in range(2):
                pltpu.make_async_copy(
                    table_hbm.at[idx_v.at[pl.ds(0, _HIST)]],
                    rows_v.at[b, u], gsems[b]).wait()

        def start_store(i, b):
            pltpu.async_copy(
                rows_v.at[b],
                out_hbm.at[pl.ds(b0 + 2 * i, 2), :, pl.ds(0, _D)], osems[b])

        def wait_store(b):
            pltpu.make_async_copy(
                rows_v.at[b],
                out_hbm.at[pl.ds(b0, 2), :, pl.ds(0, _D)], osems[b]).wait()

        for b in range(_NBUF):
            start_gather(b, b)

        @pl.loop(0, _NGRP)
        def _grp(g):
            i0 = g * _NBUF
            for b in range(_NBUF):
                wait_gather(b)
                start_store(i0 + b, b)
            for b in range(_NBUF):
                wait_store(b)
                start_gather(i0 + b + _NBUF, b)

        i0 = _NGRP * _NBUF
        for b in range(_NBUF):
            wait_gather(b)
            start_store(i0 + b, b)
        for b in range(_NBUF):
            wait_store(b)

    return emb


_emb = _make_kernel()


@jax.jit
def kernel(x, table):
    idx = x.reshape(_B).astype(jnp.int32) * 2
    tab2m = jnp.pad(table, ((0, 0), (0, _DP - _D))).reshape(2 * _V, _D)
    out128 = _emb(idx, tab2m)
    return out128[:, :, :_D]


# speedup vs baseline: 2.0499x; 1.0017x over previous
import functools

import jax
import jax.numpy as jnp
from jax import lax
from jax.experimental import pallas as pl
from jax.experimental.pallas import tpu as pltpu
from jax.experimental.pallas import tpu_sc as plsc

_BATCH = 4096
_HIST = 200
_D = 64
_DP = 128
_V = 1000000
_B = _BATCH * _HIST
_NC = 2
_NS = 16
_NW = _NC * _NS
_BPW = _B // _NW
_CH = _HIST
_NBUF = 8
_NCHUNK = _BPW // _CH
_NGRP = _NCHUNK // _NBUF - 1


def _make_kernel():
    mesh = plsc.VectorSubcoreMesh(core_axis_name="c", subcore_axis_name="s")

    @functools.partial(
        pl.kernel,
        out_type=jax.ShapeDtypeStruct((_BATCH, _HIST, _DP), jnp.float32),
        mesh=mesh,
        scratch_types=(
            [pltpu.VMEM((_BPW,), jnp.int32),
             pltpu.VMEM((_NBUF, _CH, _D), jnp.float32)]
            + [pltpu.SemaphoreType.DMA] * (2 * _NBUF)
        ),
        compiler_params=pltpu.CompilerParams(use_tc_tiling_on_sc=False),
    )
    def emb(idx_hbm, table_hbm, out_hbm, idx_v, rows_v, *sems):
        gsems = sems[:_NBUF]
        osems = sems[_NBUF:]
        wid = lax.axis_index("s") * _NC + lax.axis_index("c")
        base = wid * _BPW
        b0 = wid * (_BATCH // _NW)

        pltpu.sync_copy(idx_hbm.at[pl.ds(base, _BPW)], idx_v)

        def start_gather(i, b):
            pltpu.async_copy(
                table_hbm.at[idx_v.at[pl.ds(i * _CH, _CH)]],
                rows_v.at[b], gsems[b])

        def wait_gather(b):
            pltpu.make_async_copy(
                table_hbm.at[idx_v.at[pl.ds(0, _CH)]],
                rows_v.at[b], gsems[b]).wait()

        def start_store(i, b):
            pltpu.async_copy(
                rows_v.at[b], out_hbm.at[b0 + i, :, pl.ds(0, _D)], osems[b])

        def wait_store(b):
            pltpu.make_async_copy(
                rows_v.at[b], out_hbm.at[b0, :, pl.ds(0, _D)], osems[b]).wait()

        for b in range(_NBUF):
            start_gather(b, b)

        @pl.loop(0, _NGRP)
        def _grp(g):
            i0 = g * _NBUF
            for b in range(_NBUF):
                wait_gather(b)
                start_store(i0 + b, b)
            for b in range(_NBUF):
                wait_store(b)
                start_gather(i0 + b + _NBUF, b)

        i0 = _NGRP * _NBUF
        for b in range(_NBUF):
            wait_gather(b)
            start_store(i0 + b, b)
        for b in range(_NBUF):
            wait_store(b)

    return emb


_emb = _make_kernel()


@jax.jit
def kernel(x, table):
    idx = x.reshape(_B).astype(jnp.int32) * 2
    tab2m = jnp.pad(table, ((0, 0), (0, _DP - _D))).reshape(2 * _V, _D)
    out128 = _emb(idx, tab2m)
    return out128[:, :, :_D]
